# baseline (speedup 1.0000x reference)
import jax
import jax.numpy as jnp
from jax.experimental import pallas as pl

N_CATEGORY = 20
N_TYPE = 200
N_VARIANT = 800
N_SPATIAL = 20
D = 320
TOTAL = N_CATEGORY + N_TYPE + N_VARIANT + N_SPATIAL


def _concat_body(cat_ref, typ_ref, var_ref, spa_ref, out_ref):
    out_ref[...] = jnp.concatenate(
        [cat_ref[...], typ_ref[...], var_ref[...], spa_ref[...]], axis=0
    )


def kernel(category_codes, type_codes, variant_codes, spatial_codes):
    typ = type_codes.reshape(N_TYPE, D)
    var = variant_codes.reshape(N_VARIANT, D)
    return pl.pallas_call(
        _concat_body,
        out_shape=jax.ShapeDtypeStruct((TOTAL, D), jnp.float32),
    )(category_codes, typ, var, spatial_codes)
